# baseline (speedup 1.0000x reference)
import jax
import jax.numpy as jnp
import numpy as np
from jax.experimental import pallas as pl
from jax.experimental.pallas import tpu as pltpu

H = 50
HP = 56
NBASIS = 8
LAYERS = 6
BAR_HALF = 0.1
MAX_RADIUS = 0.06
GROWS = 8


def _fused_kernel(yt_ref, We_ref, Wm_ref, bm_ref, Wu_ref, bu_ref, Wo_ref,
                  bo_ref, out_ref, inb):
    N = yt_ref.shape[1]
    L = N // GROWS
    f32 = jnp.float32
    cax = yt_ref[0:1, :]
    cay = yt_ref[1:2, :]
    a1 = yt_ref[2:3, :]

    prevx = jnp.concatenate([cax[:, 0:1], cax[:, :-1]], axis=1)
    prevy = jnp.concatenate([cay[:, 0:1], cay[:, :-1]], axis=1)

    dx = BAR_HALF * jnp.cos(a1)
    dy = BAR_HALF * jnp.sin(a1)
    ex = cax - prevx
    ey = cay - prevy
    elen6 = jnp.sqrt(ex * ex + ey * ey + 1e-12)
    elen2 = jnp.sqrt(dx * dx + dy * dy + 1e-12)

    centers = [float(c) for c in
               np.linspace(0.0, MAX_RADIUS, NBASIS).astype(np.float32)]
    inv = float(np.float32(NBASIS / MAX_RADIUS))
    basis6 = jnp.concatenate(
        [jnp.exp(-jnp.square((elen6 - c) * inv)) for c in centers], axis=0)
    basis2 = jnp.concatenate(
        [jnp.exp(-jnp.square((elen2 - c) * inv)) for c in centers], axis=0)

    one = jnp.ones((1, N), f32)
    zero = jnp.zeros((1, N), f32)
    z2 = jnp.zeros((2, N), f32)
    ef6 = jnp.concatenate([one, zero, basis6, ex, ey, zero, one, z2], axis=0)
    ef2 = jnp.concatenate([zero, one, basis2, dx, dy, zero, one, z2], axis=0)
    ef3 = jnp.concatenate([zero, one, basis2, -dx, -dy, zero, one, z2],
                          axis=0)
    inb[112:128, 0:N] = ef2
    inb[112:128, N:2 * N] = ef6
    inb[112:128, 2 * N:3 * N] = ef3

    def zc(*s):
        return jnp.zeros(s, f32)

    WeT = jnp.transpose(We_ref[...])
    hc = jnp.concatenate([WeT[:, 0:1] + WeT[:, 1:2], zc(HP - H, 1)], axis=0)
    he = jnp.concatenate([WeT[:, 0:1] + WeT[:, 2:3], zc(HP - H, 1)], axis=0)

    Wmp = []
    Wup = []
    buc = []
    for l in range(LAYERS):
        Tl = jnp.transpose(jnp.concatenate(
            [Wm_ref[l], Wu_ref[l], bm_ref[l], bu_ref[l]], axis=0))
        wm = jnp.concatenate(
            [Tl[:, 0:H], zc(H, 6), Tl[:, H:2 * H], zc(H, 6),
             Tl[:, 2 * H:113], Tl[:, 163:164], zc(H, 2)], axis=1)
        Wmp.append(jnp.concatenate([wm, zc(HP - H, 128)], axis=0))
        wu = jnp.concatenate([Tl[:, 113:163], zc(H, 6)], axis=1)
        Wup.append(jnp.concatenate([wu, zc(HP - H, HP)], axis=0))
        buc.append(jnp.concatenate([Tl[:, 164:165], zc(HP - H, 1)], axis=0))

    WoT = jnp.transpose(Wo_ref[...])
    Wop = jnp.concatenate([
        jnp.concatenate([WoT, zc(3, HP - H)], axis=1), zc(5, HP)], axis=0)
    boc = jnp.concatenate([jnp.transpose(bo_ref[...]), zc(5, 1)], axis=0)

    zN = jnp.zeros((HP, N), f32)
    inb[0:56, 0:N] = hc + zN
    inb[0:56, N:2 * N] = hc + zN
    inb[0:56, 2 * N:3 * N] = hc + zN
    inb[56:112, 0:N] = he + zN
    inb[56:112, N:2 * N] = hc + zN
    inb[56:112, 2 * N:3 * N] = he + zN
    h1 = hc

    def dot(a, b):
        return jnp.dot(a, b, preferred_element_type=f32)

    g = jax.nn.gelu
    Snew = None
    for l in range(LAYERS):
        IN = inb[...]
        M = g(dot(Wmp[l], IN))
        S = IN[56:112, :]
        Snew = g(dot(Wup[l], S + M) + buc[l])
        if l < LAYERS - 1:
            h1 = g(dot(Wup[l], h1) + buc[l])
            inb[56:112, :] = Snew
            H4n = Snew[:, N:2 * N]
            inb[0:56, 0:N] = H4n
            inb[0:56, N:2 * N] = h1 + zN
            inb[0:56, 2 * N:3 * N] = H4n

    O = dot(Wop, Snew) + boc
    o3 = O[:, 0:N]
    o4 = O[:, N:2 * N]
    o5 = O[:, 2 * N:3 * N]
    fx = o3[0:1] + o4[0:1] + o5[0:1]
    fy = o3[1:2] + o4[1:2] + o5[1:2]
    tq = dx * (o3[1:2] - o5[1:2]) - dy * (o3[0:1] - o5[0:1])
    res = jnp.concatenate([fx, fy, tq], axis=0)

    lane = jax.lax.broadcasted_iota(jnp.int32, (1, N), 1) % L
    mask = (lane >= 1) & (lane <= L - 2)
    out_ref[...] = jnp.where(mask, res, 0.0)


def kernel(y, W_embed, W_msg, b_msg, W_upd, b_upd, W_out, b_out):
    B, L, _ = y.shape
    f32 = jnp.float32
    yt = y.transpose(2, 0, 1).reshape(6, B * L)
    NL = GROWS * L

    def full(a):
        return pl.BlockSpec(a.shape, lambda g, _n=a.ndim: (0,) * _n)

    bm3 = b_msg.reshape(LAYERS, 1, H)
    bu3 = b_upd.reshape(LAYERS, 1, H)
    bo2 = b_out.reshape(1, 3)

    out = pl.pallas_call(
        _fused_kernel,
        grid=(B // GROWS,),
        in_specs=[
            pl.BlockSpec((6, NL), lambda g: (0, g)),
            full(W_embed), full(W_msg), full(bm3), full(W_upd), full(bu3),
            full(W_out), full(bo2),
        ],
        out_specs=pl.BlockSpec((3, NL), lambda g: (0, g)),
        out_shape=jax.ShapeDtypeStruct((3, B * L), f32),
        scratch_shapes=[pltpu.VMEM((128, 3 * NL), f32)],
        compiler_params=pltpu.CompilerParams(
            dimension_semantics=("arbitrary",)),
    )(yt, W_embed, W_msg, bm3, W_upd, bu3, W_out, bo2)
    return out.reshape(3, B, L).transpose(1, 2, 0)


# speedup vs baseline: 1.2881x; 1.2881x over previous
import jax
import jax.numpy as jnp
import numpy as np
from jax.experimental import pallas as pl
from jax.experimental.pallas import tpu as pltpu

H = 50
HP = 56
NBASIS = 8
LAYERS = 6
BAR_HALF = 0.1
MAX_RADIUS = 0.06
GROWS = 2


def _fused_kernel(yt_ref, We_ref, Wm_ref, bm_ref, Wu_ref, bu_ref, Wo_ref,
                  bo_ref, out_ref, inb):
    N = yt_ref.shape[1]
    L = N // GROWS
    f32 = jnp.float32
    cax = yt_ref[0:1, :]
    cay = yt_ref[1:2, :]
    a1 = yt_ref[2:3, :]

    prevx = jnp.concatenate([cax[:, 0:1], cax[:, :-1]], axis=1)
    prevy = jnp.concatenate([cay[:, 0:1], cay[:, :-1]], axis=1)

    dx = BAR_HALF * jnp.cos(a1)
    dy = BAR_HALF * jnp.sin(a1)
    ex = cax - prevx
    ey = cay - prevy
    elen6 = jnp.sqrt(ex * ex + ey * ey + 1e-12)
    elen2 = jnp.sqrt(dx * dx + dy * dy + 1e-12)

    centers = [float(c) for c in
               np.linspace(0.0, MAX_RADIUS, NBASIS).astype(np.float32)]
    inv = float(np.float32(NBASIS / MAX_RADIUS))
    basis6 = jnp.concatenate(
        [jnp.exp(-jnp.square((elen6 - c) * inv)) for c in centers], axis=0)
    basis2 = jnp.concatenate(
        [jnp.exp(-jnp.square((elen2 - c) * inv)) for c in centers], axis=0)

    one = jnp.ones((1, N), f32)
    zero = jnp.zeros((1, N), f32)
    z2 = jnp.zeros((2, N), f32)
    ef6 = jnp.concatenate([one, zero, basis6, ex, ey, zero, one, z2], axis=0)
    ef2 = jnp.concatenate([zero, one, basis2, dx, dy, zero, one, z2], axis=0)
    ef3 = jnp.concatenate([zero, one, basis2, -dx, -dy, zero, one, z2],
                          axis=0)
    inb[112:128, 0:N] = ef2
    inb[112:128, N:2 * N] = ef6
    inb[112:128, 2 * N:3 * N] = ef3

    def zc(*s):
        return jnp.zeros(s, f32)

    WeT = jnp.transpose(We_ref[...])
    hc = jnp.concatenate([WeT[:, 0:1] + WeT[:, 1:2], zc(HP - H, 1)], axis=0)
    he = jnp.concatenate([WeT[:, 0:1] + WeT[:, 2:3], zc(HP - H, 1)], axis=0)

    Wmp = []
    Wup = []
    buc = []
    for l in range(LAYERS):
        Tl = jnp.transpose(jnp.concatenate(
            [Wm_ref[l], Wu_ref[l], bm_ref[l], bu_ref[l]], axis=0))
        wm = jnp.concatenate(
            [Tl[:, 0:H], zc(H, 6), Tl[:, H:2 * H], zc(H, 6),
             Tl[:, 2 * H:113], Tl[:, 163:164], zc(H, 2)], axis=1)
        Wmp.append(jnp.concatenate([wm, zc(HP - H, 128)], axis=0))
        wu = jnp.concatenate([Tl[:, 113:163], zc(H, 6)], axis=1)
        Wup.append(jnp.concatenate([wu, zc(HP - H, HP)], axis=0))
        buc.append(jnp.concatenate([Tl[:, 164:165], zc(HP - H, 1)], axis=0))

    WoT = jnp.transpose(Wo_ref[...])
    Wop = jnp.concatenate([
        jnp.concatenate([WoT, zc(3, HP - H)], axis=1), zc(5, HP)], axis=0)
    boc = jnp.concatenate([jnp.transpose(bo_ref[...]), zc(5, 1)], axis=0)

    zN = jnp.zeros((HP, N), f32)
    inb[0:56, 0:N] = hc + zN
    inb[0:56, N:2 * N] = hc + zN
    inb[0:56, 2 * N:3 * N] = hc + zN
    inb[56:112, 0:N] = he + zN
    inb[56:112, N:2 * N] = hc + zN
    inb[56:112, 2 * N:3 * N] = he + zN
    h1 = hc

    def dot(a, b):
        return jnp.dot(a, b, preferred_element_type=f32)

    g = jax.nn.gelu
    Snew = None
    for l in range(LAYERS):
        IN = inb[...]
        M = g(dot(Wmp[l], IN))
        S = IN[56:112, :]
        Snew = g(dot(Wup[l], S + M) + buc[l])
        if l < LAYERS - 1:
            h1 = g(dot(Wup[l], h1) + buc[l])
            inb[56:112, :] = Snew
            H4n = Snew[:, N:2 * N]
            inb[0:56, 0:N] = H4n
            inb[0:56, N:2 * N] = h1 + zN
            inb[0:56, 2 * N:3 * N] = H4n

    O = dot(Wop, Snew) + boc
    o3 = O[:, 0:N]
    o4 = O[:, N:2 * N]
    o5 = O[:, 2 * N:3 * N]
    fx = o3[0:1] + o4[0:1] + o5[0:1]
    fy = o3[1:2] + o4[1:2] + o5[1:2]
    tq = dx * (o3[1:2] - o5[1:2]) - dy * (o3[0:1] - o5[0:1])
    res = jnp.concatenate([fx, fy, tq], axis=0)

    lane = jax.lax.broadcasted_iota(jnp.int32, (1, N), 1) % L
    mask = (lane >= 1) & (lane <= L - 2)
    out_ref[...] = jnp.where(mask, res, 0.0)


def kernel(y, W_embed, W_msg, b_msg, W_upd, b_upd, W_out, b_out):
    B, L, _ = y.shape
    f32 = jnp.float32
    yt = y.transpose(2, 0, 1).reshape(6, B * L)
    NL = GROWS * L

    def full(a):
        return pl.BlockSpec(a.shape, lambda g, _n=a.ndim: (0,) * _n)

    bm3 = b_msg.reshape(LAYERS, 1, H)
    bu3 = b_upd.reshape(LAYERS, 1, H)
    bo2 = b_out.reshape(1, 3)

    out = pl.pallas_call(
        _fused_kernel,
        grid=(B // GROWS,),
        in_specs=[
            pl.BlockSpec((6, NL), lambda g: (0, g)),
            full(W_embed), full(W_msg), full(bm3), full(W_upd), full(bu3),
            full(W_out), full(bo2),
        ],
        out_specs=pl.BlockSpec((3, NL), lambda g: (0, g)),
        out_shape=jax.ShapeDtypeStruct((3, B * L), f32),
        scratch_shapes=[pltpu.VMEM((128, 3 * NL), f32)],
        compiler_params=pltpu.CompilerParams(
            dimension_semantics=("arbitrary",)),
    )(yt, W_embed, W_msg, bm3, W_upd, bu3, W_out, bo2)
    return out.reshape(3, B, L).transpose(1, 2, 0)


# speedup vs baseline: 1.4151x; 1.0986x over previous
import jax
import jax.numpy as jnp
import numpy as np
from jax.experimental import pallas as pl
from jax.experimental.pallas import tpu as pltpu

H = 50
HP = 56
NBASIS = 8
LAYERS = 6
BAR_HALF = 0.1
MAX_RADIUS = 0.06
GROWS = 4


def _fused_kernel(yt_ref, We_ref, Wm_ref, bm_ref, Wu_ref, bu_ref, Wo_ref,
                  bo_ref, out_ref, inb):
    N = yt_ref.shape[1]
    L = N // GROWS
    f32 = jnp.float32
    cax = yt_ref[0:1, :]
    cay = yt_ref[1:2, :]
    a1 = yt_ref[2:3, :]

    prevx = jnp.concatenate([cax[:, 0:1], cax[:, :-1]], axis=1)
    prevy = jnp.concatenate([cay[:, 0:1], cay[:, :-1]], axis=1)

    dx = BAR_HALF * jnp.cos(a1)
    dy = BAR_HALF * jnp.sin(a1)
    ex = cax - prevx
    ey = cay - prevy
    elen6 = jnp.sqrt(ex * ex + ey * ey + 1e-12)
    elen2 = jnp.sqrt(dx * dx + dy * dy + 1e-12)

    centers = [float(c) for c in
               np.linspace(0.0, MAX_RADIUS, NBASIS).astype(np.float32)]
    inv = float(np.float32(NBASIS / MAX_RADIUS))
    basis6 = jnp.concatenate(
        [jnp.exp(-jnp.square((elen6 - c) * inv)) for c in centers], axis=0)
    basis2 = jnp.concatenate(
        [jnp.exp(-jnp.square((elen2 - c) * inv)) for c in centers], axis=0)

    one = jnp.ones((1, N), f32)
    zero = jnp.zeros((1, N), f32)
    z2 = jnp.zeros((2, N), f32)
    ef6 = jnp.concatenate([one, zero, basis6, ex, ey, zero, one, z2], axis=0)
    ef2 = jnp.concatenate([zero, one, basis2, dx, dy, zero, one, z2], axis=0)
    ef3 = jnp.concatenate([zero, one, basis2, -dx, -dy, zero, one, z2],
                          axis=0)
    inb[112:128, 0:N] = ef2
    inb[112:128, N:2 * N] = ef6
    inb[112:128, 2 * N:3 * N] = ef3

    def zc(*s):
        return jnp.zeros(s, f32)

    WeT = jnp.transpose(We_ref[...])
    hc = jnp.concatenate([WeT[:, 0:1] + WeT[:, 1:2], zc(HP - H, 1)], axis=0)
    he = jnp.concatenate([WeT[:, 0:1] + WeT[:, 2:3], zc(HP - H, 1)], axis=0)

    Wmp = []
    Wup = []
    buc = []
    for l in range(LAYERS):
        Tl = jnp.transpose(jnp.concatenate(
            [Wm_ref[l], Wu_ref[l], bm_ref[l], bu_ref[l]], axis=0))
        wm = jnp.concatenate(
            [Tl[:, 0:H], zc(H, 6), Tl[:, H:2 * H], zc(H, 6),
             Tl[:, 2 * H:113], Tl[:, 163:164], zc(H, 2)], axis=1)
        sent = jnp.where(
            jax.lax.broadcasted_iota(jnp.int32, (1, 128), 1) == 125,
            16.0, 0.0)
        Wmp.append(jnp.concatenate([wm, zc(5, 128), sent], axis=0))
        wu = jnp.concatenate(
            [Tl[:, 113:163], zc(H, 5), Tl[:, 164:165] * 0.0625], axis=1)
        Wup.append(jnp.concatenate([wu, zc(HP - H, HP)], axis=0))
        buc.append(jnp.concatenate([Tl[:, 164:165], zc(HP - H, 1)], axis=0))

    WoT = jnp.transpose(Wo_ref[...])
    Wop = jnp.concatenate([
        jnp.concatenate([WoT, zc(3, HP - H)], axis=1), zc(5, HP)], axis=0)
    boc = jnp.concatenate([jnp.transpose(bo_ref[...]), zc(5, 1)], axis=0)

    zN = jnp.zeros((HP, N), f32)
    inb[0:56, 0:N] = hc + zN
    inb[0:56, N:2 * N] = hc + zN
    inb[0:56, 2 * N:3 * N] = hc + zN
    inb[56:112, 0:N] = he + zN
    inb[56:112, N:2 * N] = hc + zN
    inb[56:112, 2 * N:3 * N] = he + zN
    h1 = hc

    def dot(a, b):
        return jnp.dot(a, b, preferred_element_type=f32)

    C1 = 0.7978845608028654
    C2 = C1 * 0.044715

    def g(x):
        u = x * (C1 + C2 * (x * x))
        return x * (0.5 + 0.5 * jnp.tanh(u))

    Snew = None
    for l in range(LAYERS):
        IN = inb[...]
        M = g(dot(Wmp[l], IN))
        S = IN[56:112, :]
        Snew = g(dot(Wup[l], S + M))
        if l < LAYERS - 1:
            h1 = g(dot(Wup[l], h1) + buc[l])
            inb[56:112, :] = Snew
            H4n = Snew[:, N:2 * N]
            inb[0:56, 0:N] = H4n
            inb[0:56, N:2 * N] = h1 + zN
            inb[0:56, 2 * N:3 * N] = H4n

    O = dot(Wop, Snew) + boc
    o3 = O[:, 0:N]
    o4 = O[:, N:2 * N]
    o5 = O[:, 2 * N:3 * N]
    fx = o3[0:1] + o4[0:1] + o5[0:1]
    fy = o3[1:2] + o4[1:2] + o5[1:2]
    tq = dx * (o3[1:2] - o5[1:2]) - dy * (o3[0:1] - o5[0:1])
    res = jnp.concatenate([fx, fy, tq], axis=0)

    lane = jax.lax.broadcasted_iota(jnp.int32, (1, N), 1) % L
    mask = (lane >= 1) & (lane <= L - 2)
    out_ref[...] = jnp.where(mask, res, 0.0)


def kernel(y, W_embed, W_msg, b_msg, W_upd, b_upd, W_out, b_out):
    B, L, _ = y.shape
    f32 = jnp.float32
    yt = y.transpose(2, 0, 1).reshape(6, B * L)
    NL = GROWS * L

    def full(a):
        return pl.BlockSpec(a.shape, lambda g, _n=a.ndim: (0,) * _n)

    bm3 = b_msg.reshape(LAYERS, 1, H)
    bu3 = b_upd.reshape(LAYERS, 1, H)
    bo2 = b_out.reshape(1, 3)

    out = pl.pallas_call(
        _fused_kernel,
        grid=(B // GROWS,),
        in_specs=[
            pl.BlockSpec((6, NL), lambda g: (0, g)),
            full(W_embed), full(W_msg), full(bm3), full(W_upd), full(bu3),
            full(W_out), full(bo2),
        ],
        out_specs=pl.BlockSpec((3, NL), lambda g: (0, g)),
        out_shape=jax.ShapeDtypeStruct((3, B * L), f32),
        scratch_shapes=[pltpu.VMEM((128, 3 * NL), f32)],
        compiler_params=pltpu.CompilerParams(
            dimension_semantics=("arbitrary",)),
    )(yt, W_embed, W_msg, bm3, W_upd, bu3, W_out, bo2)
    return out.reshape(3, B, L).transpose(1, 2, 0)
